# baseline (speedup 1.0000x reference)
import functools

import jax
import jax.numpy as jnp
from jax import lax
from jax.experimental import pallas as pl
from jax.experimental.pallas import tpu as pltpu
from jax.experimental.pallas import tpu_sc as plsc

N = 10000
DH = 128
E = 320000
NC = 2
NS = 16
NW = NC * NS
E_W = E // NW
CH = 80
NCHUNK = E_W // CH

_mesh = plsc.VectorSubcoreMesh(core_axis_name="c", subcore_axis_name="s")
_f32 = jnp.float32


def _gat_edge_sc(hs, ev128, src, dst, zeros_h):
    @functools.partial(
        pl.kernel,
        out_type=jax.ShapeDtypeStruct((NC, N, DH), _f32),
        mesh=_mesh,
        scratch_types=[
            pltpu.VMEM((CH,), jnp.int32),
            pltpu.VMEM((CH,), jnp.int32),
            pltpu.VMEM((CH, DH), _f32),
            pltpu.VMEM((16, DH), _f32),
            pltpu.VMEM_SHARED((N, DH), _f32),
            pltpu.SemaphoreType.DMA,
        ],
    )
    def k(hs_h, ev_h, src_h, dst_h, z_h, nacc_o,
          idx_s, idx_d, rows, evbuf, acc, sem):
        c = lax.axis_index("c")
        s = lax.axis_index("s")
        w = c * NS + s
        ebase = w * E_W

        @pl.when(s < 10)
        def _():
            pltpu.sync_copy(z_h, acc.at[pl.ds(s * 1000, 1000)])
        plsc.subcore_barrier()

        def body(kk, _):
            base = ebase + kk * CH
            pltpu.sync_copy(src_h.at[pl.ds(base, CH)], idx_s)
            pltpu.sync_copy(dst_h.at[pl.ds(base, CH)], idx_d)
            pltpu.async_copy(hs_h.at[idx_s], rows, sem).wait()
            pltpu.sync_copy(ev_h.at[w * NCHUNK + kk], evbuf)

            def scale(q, _):
                for b in range(8):
                    evv = evbuf[q, pl.ds(b * 16, 16)]
                    r = q * 8 + b
                    for u in range(DH // 16):
                        rows[r, pl.ds(u * 16, 16)] = (
                            rows[r, pl.ds(u * 16, 16)] * evv)
                return 0
            lax.fori_loop(0, CH // 8, scale, 0)

            pltpu.sync_copy(rows, acc.at[idx_d], add=True)
            return 0
        lax.fori_loop(0, NCHUNK, body, 0)
        plsc.subcore_barrier()

        @pl.when(s < 10)
        def _():
            r0 = s * 1000
            pltpu.sync_copy(acc.at[pl.ds(r0, 1000)], nacc_o.at[c, pl.ds(r0, 1000)])

    return k(hs, ev128, src, dst, zeros_h)


def _sage_edge_sc(x_src, src, dst, zeros_h):
    @functools.partial(
        pl.kernel,
        out_type=jax.ShapeDtypeStruct((NC, N, DH), _f32),
        mesh=_mesh,
        scratch_types=[
            pltpu.VMEM((CH,), jnp.int32),
            pltpu.VMEM((CH,), jnp.int32),
            pltpu.VMEM((CH, DH), _f32),
            pltpu.VMEM_SHARED((N, DH), _f32),
            pltpu.SemaphoreType.DMA,
        ],
    )
    def k(x_h, src_h, dst_h, z_h, sum_o, idx_s, idx_d, rows, acc, sem):
        c = lax.axis_index("c")
        s = lax.axis_index("s")
        ebase = (c * NS + s) * E_W

        @pl.when(s < 10)
        def _():
            pltpu.sync_copy(z_h, acc.at[pl.ds(s * 1000, 1000)])
        plsc.subcore_barrier()

        def body(kk, _):
            base = ebase + kk * CH
            pltpu.sync_copy(src_h.at[pl.ds(base, CH)], idx_s)
            pltpu.sync_copy(dst_h.at[pl.ds(base, CH)], idx_d)
            pltpu.async_copy(x_h.at[idx_s], rows, sem).wait()
            pltpu.sync_copy(rows, acc.at[idx_d], add=True)
            return 0
        lax.fori_loop(0, NCHUNK, body, 0)
        plsc.subcore_barrier()

        @pl.when(s < 10)
        def _():
            r0 = s * 1000
            pltpu.sync_copy(acc.at[pl.ds(r0, 1000)], sum_o.at[c, pl.ds(r0, 1000)])

    return k(x_src, src, dst, zeros_h)


_BM = 1024
_GRID = (N + _BM - 1) // _BM


def _proj_tc(xm, xd, p_to, p_tr):
    def body(xm_r, xd_r, ws_to_r, wd_to_r, ats_to_r, atd_to_r,
             ws_tr_r, wd_tr_r, ats_tr_r, atd_tr_r,
             hs_to_o, asrc_to_o, adst_to_o, hs_tr_o, asrc_tr_o, adst_tr_o):
        xm_b = xm_r[...]
        xd_b = xd_r[...]
        for ws_r, wd_r, ats_r, atd_r, hs_o, as_o, ad_o in (
            (ws_to_r, wd_to_r, ats_to_r, atd_to_r, hs_to_o, asrc_to_o, adst_to_o),
            (ws_tr_r, wd_tr_r, ats_tr_r, atd_tr_r, hs_tr_o, asrc_tr_o, adst_tr_o),
        ):
            hs = jnp.dot(xm_b, ws_r[...], preferred_element_type=_f32)
            hs_o[...] = hs
            as_o[...] = hs @ ats_r[...]
            vd = wd_r[...] @ atd_r[...]
            ad_o[...] = xd_b @ vd

    full = pl.BlockSpec((DH, DH), lambda i: (0, 0))
    vec = pl.BlockSpec((DH,), lambda i: (0,))
    rows = pl.BlockSpec((_BM, DH), lambda i: (i, 0))
    rvec = pl.BlockSpec((_BM,), lambda i: (i,))
    return pl.pallas_call(
        body,
        grid=(_GRID,),
        in_specs=[rows, rows, full, full, vec, vec, full, full, vec, vec],
        out_specs=[rows, rvec, rvec, rows, rvec, rvec],
        out_shape=[
            jax.ShapeDtypeStruct((N, DH), _f32),
            jax.ShapeDtypeStruct((N,), _f32),
            jax.ShapeDtypeStruct((N,), _f32),
            jax.ShapeDtypeStruct((N, DH), _f32),
            jax.ShapeDtypeStruct((N,), _f32),
            jax.ShapeDtypeStruct((N,), _f32),
        ],
    )(xm, xd, p_to["Ws"], p_to["Wd"], p_to["att_s"], p_to["att_d"],
      p_tr["Ws"], p_tr["Wd"], p_tr["att_s"], p_tr["att_d"])


def _epilogue_tc(gat_to, den_to, gat_tr, den_tr, ssim, csim, srev, crev,
                 xm, xd, p_to, p_tr, p_sim, p_rev):
    def body(nto_r, dto_r, ntr_r, dtr_r, ssim_r, csim_r, srev_r, crev_r,
             xm_r, xd_r,
             wl_sim_r, wr_sim_r, bl_sim_r, wl_rev_r, wr_rev_r, bl_rev_r,
             b_to_r, b_tr_r, xd_o, xm_o):
        def gat(n_r, d_r, b):
            num = n_r[0] + n_r[1]
            den = d_r[...] + 1e-16
            return num / den[:, None] + b

        def sage(s_r, c_r, x_b, wl, bl, wr):
            cnt = jnp.clip(c_r[...], 1.0)
            mean = (s_r[0] + s_r[1]) / cnt[:, None]
            return (jnp.dot(mean, wl, preferred_element_type=_f32) + bl
                    + jnp.dot(x_b, wr, preferred_element_type=_f32))

        g_to = gat(nto_r, dto_r, b_to_r[...])
        g_tr = gat(ntr_r, dtr_r, b_tr_r[...])
        s_sim = sage(ssim_r, csim_r, xd_r[...], wl_sim_r[...], bl_sim_r[...],
                     wr_sim_r[...])
        s_rev = sage(srev_r, crev_r, xm_r[...], wl_rev_r[...], bl_rev_r[...],
                     wr_rev_r[...])
        xd_o[...] = jnp.maximum(g_to + s_sim + g_tr, 0.0)
        xm_o[...] = jnp.maximum(s_rev, 0.0)

    full = pl.BlockSpec((DH, DH), lambda i: (0, 0))
    vec = pl.BlockSpec((DH,), lambda i: (0,))
    rows = pl.BlockSpec((_BM, DH), lambda i: (i, 0))
    rvec = pl.BlockSpec((_BM,), lambda i: (i,))
    p_rows = pl.BlockSpec((NC, _BM, DH), lambda i: (0, i, 0))
    return pl.pallas_call(
        body,
        grid=(_GRID,),
        in_specs=[p_rows, rvec, p_rows, rvec, p_rows, rvec, p_rows, rvec,
                  rows, rows,
                  full, full, vec, full, full, vec, vec, vec],
        out_specs=[rows, rows],
        out_shape=[
            jax.ShapeDtypeStruct((N, DH), _f32),
            jax.ShapeDtypeStruct((N, DH), _f32),
        ],
    )(gat_to, den_to, gat_tr, den_tr, ssim, csim, srev, crev, xm, xd,
      p_sim["Wl"], p_sim["Wr"], p_sim["bl"], p_rev["Wl"], p_rev["Wr"],
      p_rev["bl"], p_to["b"], p_tr["b"])


def _edge_softmax_prep(asrc, adst, src, dst):
    alpha = jax.nn.leaky_relu(asrc[src] + adst[dst], 0.2)
    m = jnp.maximum(jnp.max(asrc) + jnp.max(adst), 0.0)
    ev = jnp.exp(alpha - m)
    den = jax.ops.segment_sum(ev, dst, num_segments=N)
    evp = jnp.pad(ev.reshape(NW * NCHUNK, CH), ((0, 0), (0, 128 - CH)))
    ev128 = jnp.broadcast_to(
        evp.reshape(NW * NCHUNK, 16, 8, 1), (NW * NCHUNK, 16, 8, 16)
    ).reshape(NW * NCHUNK, 16, DH)
    return ev128, den


def kernel(x_model, x_dataset, edge_index_trained_on, edge_index_similar_to,
           edge_index_rev_trained_on, edge_index_transfer_to, params):
    xm, xd = x_model, x_dataset
    zeros_h = jnp.zeros((1000, DH), _f32)
    ones_e = jnp.ones((E,), _f32)
    ei_to = edge_index_trained_on
    ei_sim = edge_index_similar_to
    ei_rev = edge_index_rev_trained_on
    ei_tr = edge_index_transfer_to
    cnt_sim = jax.ops.segment_sum(ones_e, ei_sim[1], num_segments=N)
    cnt_rev = jax.ops.segment_sum(ones_e, ei_rev[1], num_segments=N)
    for l in range(2):
        p_to = params["l%d_to" % l]
        p_tr = params["l%d_tr" % l]
        p_sim = params["l%d_sim" % l]
        p_rev = params["l%d_rev" % l]
        hs_to, asrc_to, adst_to, hs_tr, asrc_tr, adst_tr = _proj_tc(
            xm, xd, p_to, p_tr)
        ev_to, den_to = _edge_softmax_prep(asrc_to, adst_to, ei_to[0], ei_to[1])
        ev_tr, den_tr = _edge_softmax_prep(asrc_tr, adst_tr, ei_tr[0], ei_tr[1])
        g_to = _gat_edge_sc(hs_to, ev_to, ei_to[0], ei_to[1], zeros_h)
        g_tr = _gat_edge_sc(hs_tr, ev_tr, ei_tr[0], ei_tr[1], zeros_h)
        s_sim = _sage_edge_sc(xd, ei_sim[0], ei_sim[1], zeros_h)
        s_rev = _sage_edge_sc(xd, ei_rev[0], ei_rev[1], zeros_h)
        xd_new, xm_new = _epilogue_tc(g_to, den_to, g_tr, den_tr,
                                      s_sim, cnt_sim, s_rev, cnt_rev,
                                      xm, xd, p_to, p_tr, p_sim, p_rev)
        xm, xd = xm_new, xd_new
    return xm, xd


# speedup vs baseline: 1.2226x; 1.2226x over previous
import functools

import jax
import jax.numpy as jnp
from jax import lax
from jax.experimental import pallas as pl
from jax.experimental.pallas import tpu as pltpu
from jax.experimental.pallas import tpu_sc as plsc

N = 10000
DH = 128
E = 320000
NC = 2
NS = 16
NW = NC * NS
E_W = E // NW
CH = 200
NCHUNK = E_W // CH
CHR = 32

_mesh = plsc.VectorSubcoreMesh(core_axis_name="c", subcore_axis_name="s")
_f32 = jnp.float32


def _gat_edge_sc(hs, ev128, src, dst, zeros_h):
    @functools.partial(
        pl.kernel,
        out_type=jax.ShapeDtypeStruct((NC, N, DH), _f32),
        mesh=_mesh,
        scratch_types=[
            pltpu.VMEM((CH,), jnp.int32),
            pltpu.VMEM((CH,), jnp.int32),
            pltpu.VMEM((CH, DH), _f32),
            pltpu.VMEM((CHR, DH), _f32),
            pltpu.VMEM_SHARED((N, DH), _f32),
            pltpu.SemaphoreType.DMA,
        ],
    )
    def k(hs_h, ev_h, src_h, dst_h, z_h, nacc_o,
          idx_s, idx_d, rows, evbuf, acc, sem):
        c = lax.axis_index("c")
        s = lax.axis_index("s")
        w = c * NS + s
        ebase = w * E_W

        @pl.when(s < 10)
        def _():
            pltpu.sync_copy(z_h, acc.at[pl.ds(s * 1000, 1000)])
        plsc.subcore_barrier()

        def body(kk, _):
            base = ebase + kk * CH
            pltpu.sync_copy(src_h.at[pl.ds(base, CH)], idx_s)
            pltpu.sync_copy(dst_h.at[pl.ds(base, CH)], idx_d)
            pltpu.async_copy(hs_h.at[idx_s], rows, sem).wait()
            pltpu.sync_copy(ev_h.at[w * NCHUNK + kk], evbuf)

            def scale(q, _):
                for b in range(8):
                    evv = evbuf[q, pl.ds(b * 16, 16)]
                    r = q * 8 + b
                    for u in range(DH // 16):
                        rows[r, pl.ds(u * 16, 16)] = (
                            rows[r, pl.ds(u * 16, 16)] * evv)
                return 0
            lax.fori_loop(0, CH // 8, scale, 0)

            pltpu.sync_copy(rows, acc.at[idx_d], add=True)
            return 0
        lax.fori_loop(0, NCHUNK, body, 0)
        plsc.subcore_barrier()

        @pl.when(s < 10)
        def _():
            r0 = s * 1000
            pltpu.sync_copy(acc.at[pl.ds(r0, 1000)], nacc_o.at[c, pl.ds(r0, 1000)])

    return k(hs, ev128, src, dst, zeros_h)


def _sage_edge_sc(x_src, src, dst, zeros_h):
    @functools.partial(
        pl.kernel,
        out_type=jax.ShapeDtypeStruct((NC, N, DH), _f32),
        mesh=_mesh,
        scratch_types=[
            pltpu.VMEM((CH,), jnp.int32),
            pltpu.VMEM((CH,), jnp.int32),
            pltpu.VMEM((CH, DH), _f32),
            pltpu.VMEM_SHARED((N, DH), _f32),
            pltpu.SemaphoreType.DMA,
        ],
    )
    def k(x_h, src_h, dst_h, z_h, sum_o, idx_s, idx_d, rows, acc, sem):
        c = lax.axis_index("c")
        s = lax.axis_index("s")
        ebase = (c * NS + s) * E_W

        @pl.when(s < 10)
        def _():
            pltpu.sync_copy(z_h, acc.at[pl.ds(s * 1000, 1000)])
        plsc.subcore_barrier()

        def body(kk, _):
            base = ebase + kk * CH
            pltpu.sync_copy(src_h.at[pl.ds(base, CH)], idx_s)
            pltpu.sync_copy(dst_h.at[pl.ds(base, CH)], idx_d)
            pltpu.async_copy(x_h.at[idx_s], rows, sem).wait()
            pltpu.sync_copy(rows, acc.at[idx_d], add=True)
            return 0
        lax.fori_loop(0, NCHUNK, body, 0)
        plsc.subcore_barrier()

        @pl.when(s < 10)
        def _():
            r0 = s * 1000
            pltpu.sync_copy(acc.at[pl.ds(r0, 1000)], sum_o.at[c, pl.ds(r0, 1000)])

    return k(x_src, src, dst, zeros_h)


_BM = 1024
_GRID = (N + _BM - 1) // _BM


def _proj_tc(xm, xd, p_to, p_tr):
    def body(xm_r, xd_r, ws_to_r, wd_to_r, ats_to_r, atd_to_r,
             ws_tr_r, wd_tr_r, ats_tr_r, atd_tr_r,
             hs_to_o, asrc_to_o, adst_to_o, hs_tr_o, asrc_tr_o, adst_tr_o):
        xm_b = xm_r[...]
        xd_b = xd_r[...]
        for ws_r, wd_r, ats_r, atd_r, hs_o, as_o, ad_o in (
            (ws_to_r, wd_to_r, ats_to_r, atd_to_r, hs_to_o, asrc_to_o, adst_to_o),
            (ws_tr_r, wd_tr_r, ats_tr_r, atd_tr_r, hs_tr_o, asrc_tr_o, adst_tr_o),
        ):
            hs = jnp.dot(xm_b, ws_r[...], preferred_element_type=_f32)
            hs_o[...] = hs
            as_o[...] = hs @ ats_r[...]
            vd = wd_r[...] @ atd_r[...]
            ad_o[...] = xd_b @ vd

    full = pl.BlockSpec((DH, DH), lambda i: (0, 0))
    vec = pl.BlockSpec((DH,), lambda i: (0,))
    rows = pl.BlockSpec((_BM, DH), lambda i: (i, 0))
    rvec = pl.BlockSpec((_BM,), lambda i: (i,))
    return pl.pallas_call(
        body,
        grid=(_GRID,),
        in_specs=[rows, rows, full, full, vec, vec, full, full, vec, vec],
        out_specs=[rows, rvec, rvec, rows, rvec, rvec],
        out_shape=[
            jax.ShapeDtypeStruct((N, DH), _f32),
            jax.ShapeDtypeStruct((N,), _f32),
            jax.ShapeDtypeStruct((N,), _f32),
            jax.ShapeDtypeStruct((N, DH), _f32),
            jax.ShapeDtypeStruct((N,), _f32),
            jax.ShapeDtypeStruct((N,), _f32),
        ],
    )(xm, xd, p_to["Ws"], p_to["Wd"], p_to["att_s"], p_to["att_d"],
      p_tr["Ws"], p_tr["Wd"], p_tr["att_s"], p_tr["att_d"])


def _epilogue_tc(gat_to, den_to, gat_tr, den_tr, ssim, csim, srev, crev,
                 xm, xd, p_to, p_tr, p_sim, p_rev):
    def body(nto_r, dto_r, ntr_r, dtr_r, ssim_r, csim_r, srev_r, crev_r,
             xm_r, xd_r,
             wl_sim_r, wr_sim_r, bl_sim_r, wl_rev_r, wr_rev_r, bl_rev_r,
             b_to_r, b_tr_r, xd_o, xm_o):
        def gat(n_r, d_r, b):
            num = n_r[0] + n_r[1]
            den = d_r[...] + 1e-16
            return num / den[:, None] + b

        def sage(s_r, c_r, x_b, wl, bl, wr):
            cnt = jnp.clip(c_r[...], 1.0)
            mean = (s_r[0] + s_r[1]) / cnt[:, None]
            return (jnp.dot(mean, wl, preferred_element_type=_f32) + bl
                    + jnp.dot(x_b, wr, preferred_element_type=_f32))

        g_to = gat(nto_r, dto_r, b_to_r[...])
        g_tr = gat(ntr_r, dtr_r, b_tr_r[...])
        s_sim = sage(ssim_r, csim_r, xd_r[...], wl_sim_r[...], bl_sim_r[...],
                     wr_sim_r[...])
        s_rev = sage(srev_r, crev_r, xm_r[...], wl_rev_r[...], bl_rev_r[...],
                     wr_rev_r[...])
        xd_o[...] = jnp.maximum(g_to + s_sim + g_tr, 0.0)
        xm_o[...] = jnp.maximum(s_rev, 0.0)

    full = pl.BlockSpec((DH, DH), lambda i: (0, 0))
    vec = pl.BlockSpec((DH,), lambda i: (0,))
    rows = pl.BlockSpec((_BM, DH), lambda i: (i, 0))
    rvec = pl.BlockSpec((_BM,), lambda i: (i,))
    p_rows = pl.BlockSpec((NC, _BM, DH), lambda i: (0, i, 0))
    return pl.pallas_call(
        body,
        grid=(_GRID,),
        in_specs=[p_rows, rvec, p_rows, rvec, p_rows, rvec, p_rows, rvec,
                  rows, rows,
                  full, full, vec, full, full, vec, vec, vec],
        out_specs=[rows, rows],
        out_shape=[
            jax.ShapeDtypeStruct((N, DH), _f32),
            jax.ShapeDtypeStruct((N, DH), _f32),
        ],
    )(gat_to, den_to, gat_tr, den_tr, ssim, csim, srev, crev, xm, xd,
      p_sim["Wl"], p_sim["Wr"], p_sim["bl"], p_rev["Wl"], p_rev["Wr"],
      p_rev["bl"], p_to["b"], p_tr["b"])


def _edge_softmax_prep(asrc, adst, src, dst):
    alpha = jax.nn.leaky_relu(asrc[src] + adst[dst], 0.2)
    ev = jnp.exp(alpha - jnp.max(alpha))
    den = jax.ops.segment_sum(ev, dst, num_segments=N)
    evp = jnp.pad(ev.reshape(NW * NCHUNK, CH), ((0, 0), (0, CHR * 8 - CH)))
    ev128 = jnp.broadcast_to(
        evp.reshape(NW * NCHUNK, CHR, 8, 1), (NW * NCHUNK, CHR, 8, 16)
    ).reshape(NW * NCHUNK, CHR, DH)
    return ev128, den


def kernel(x_model, x_dataset, edge_index_trained_on, edge_index_similar_to,
           edge_index_rev_trained_on, edge_index_transfer_to, params):
    xm, xd = x_model, x_dataset
    zeros_h = jnp.zeros((1000, DH), _f32)
    ones_e = jnp.ones((E,), _f32)
    ei_to = edge_index_trained_on
    ei_sim = edge_index_similar_to
    ei_rev = edge_index_rev_trained_on
    ei_tr = edge_index_transfer_to
    cnt_sim = jax.ops.segment_sum(ones_e, ei_sim[1], num_segments=N)
    cnt_rev = jax.ops.segment_sum(ones_e, ei_rev[1], num_segments=N)
    for l in range(2):
        p_to = params["l%d_to" % l]
        p_tr = params["l%d_tr" % l]
        p_sim = params["l%d_sim" % l]
        p_rev = params["l%d_rev" % l]
        hs_to, asrc_to, adst_to, hs_tr, asrc_tr, adst_tr = _proj_tc(
            xm, xd, p_to, p_tr)
        ev_to, den_to = _edge_softmax_prep(asrc_to, adst_to, ei_to[0], ei_to[1])
        ev_tr, den_tr = _edge_softmax_prep(asrc_tr, adst_tr, ei_tr[0], ei_tr[1])
        g_to = _gat_edge_sc(hs_to, ev_to, ei_to[0], ei_to[1], zeros_h)
        g_tr = _gat_edge_sc(hs_tr, ev_tr, ei_tr[0], ei_tr[1], zeros_h)
        s_sim = _sage_edge_sc(xd, ei_sim[0], ei_sim[1], zeros_h)
        s_rev = _sage_edge_sc(xd, ei_rev[0], ei_rev[1], zeros_h)
        xd_new, xm_new = _epilogue_tc(g_to, den_to, g_tr, den_tr,
                                      s_sim, cnt_sim, s_rev, cnt_rev,
                                      xm, xd, p_to, p_tr, p_sim, p_rev)
        xm, xd = xm_new, xd_new
    return xm, xd
